# baseline (speedup 1.0000x reference)
import jax
import jax.numpy as jnp
from jax import lax
from jax.experimental import pallas as pl
from jax.experimental.pallas import tpu as pltpu
from jax.experimental.pallas import tpu_sc as plsc

B = 16
N = 131072
SH = 64
NBINS = SH * SH * SH
Q = 4
QB = NBINS // Q
PCH = 8192
NCH = N // PCH
XCH = 8192

NC = 2
NS = 16
NW = NC * NS


def _sc_body(z_hbm, x_hbm, counts_hbm, r_hbm, zbuf, hist, xbuf, rbuf):
    c = lax.axis_index("c")
    s = lax.axis_index("s")
    wid = s * NC + c
    lane3 = lax.iota(jnp.int32, 16) * 3
    ones = jnp.ones((16,), jnp.float32)

    for rep in range(2):
        pair = wid + rep * NW
        b = pair // Q
        q = pair % Q
        qlo = q * QB

        @pl.loop(0, QB // 16, unroll=8)
        def _(i):
            hist[pl.ds(i * 16, 16)] = jnp.zeros((16,), jnp.float32)

        @pl.loop(0, NCH)
        def _(ch):
            pltpu.sync_copy(
                z_hbm.at[pl.ds(b * (N * 3) + ch * (PCH * 3), PCH * 3)], zbuf)

            @pl.loop(0, PCH // 16, unroll=2)
            def _(i):
                i0 = lane3 + i * 48
                v0 = plsc.load_gather(zbuf, [i0])
                v1 = plsc.load_gather(zbuf, [i0 + 1])
                v2 = plsc.load_gather(zbuf, [i0 + 2])
                w0 = ((v0 * 64.0) * 0.9999999).astype(jnp.int32)
                w1 = ((v1 * 64.0) * 0.9999999).astype(jnp.int32)
                w2 = ((v2 * 64.0) * 0.9999999).astype(jnp.int32)
                lin = (w0 << 12) + (w1 << 6) + w2
                loc = lin - qlo
                m = (loc >= 0) & (loc < QB)
                locc = jnp.clip(loc, 0, QB - 1)
                plsc.addupdate_scatter(hist, [locc], ones, mask=m)

        base_out = b * NBINS + qlo
        pltpu.sync_copy(hist, counts_hbm.at[pl.ds(base_out, QB)])

        @pl.loop(0, QB // XCH)
        def _(t):
            off = base_out + t * XCH
            pltpu.sync_copy(x_hbm.at[pl.ds(off, XCH)], xbuf)

            @pl.loop(0, XCH // 16, unroll=4)
            def _(j):
                xv = xbuf[pl.ds(j * 16, 16)]
                hv = hist[pl.ds(t * XCH + j * 16, 16)]
                rbuf[pl.ds(j * 16, 16)] = jnp.where(hv > 0.0, xv, 0.0)

            pltpu.sync_copy(rbuf, r_hbm.at[pl.ds(off, XCH)])


@jax.jit
def kernel(x, z):
    xf = x.reshape(B * NBINS)
    zf = z.reshape(B * N * 3)
    mesh = plsc.VectorSubcoreMesh(core_axis_name="c", subcore_axis_name="s")
    counts_f, r_f = pl.kernel(
        _sc_body,
        out_type=(
            jax.ShapeDtypeStruct((B * NBINS,), jnp.float32),
            jax.ShapeDtypeStruct((B * NBINS,), jnp.float32),
        ),
        mesh=mesh,
        compiler_params=pltpu.CompilerParams(needs_layout_passes=False),
        scratch_types=[
            pltpu.VMEM((PCH * 3,), jnp.float32),
            pltpu.VMEM((QB,), jnp.float32),
            pltpu.VMEM((XCH,), jnp.float32),
            pltpu.VMEM((XCH,), jnp.float32),
        ],
    )(zf, xf)
    return (counts_f.reshape(B, SH, SH, SH), r_f.reshape(B, SH, SH, SH))
